# baseline (speedup 1.0000x reference)
import functools

import jax
import jax.numpy as jnp
from jax import lax
from jax.experimental import pallas as pl
from jax.experimental.pallas import tpu as pltpu
from jax.experimental.pallas import tpu_sc as plsc

_VOCAB = 1000
_ROWS = 1024 * 50
_NC, _NS, _L = 2, 16, 16
_NW = _NC * _NS
_RPW = _ROWS // _NW
_G = 32
_BLK = _G * _VOCAB
_STEPS = _RPW // _G

_mesh = plsc.VectorSubcoreMesh(
    core_axis_name="c", subcore_axis_name="s",
    num_cores=_NC, num_subcores=_NS,
)


@functools.partial(
    pl.kernel,
    out_type=jax.ShapeDtypeStruct((_ROWS * _VOCAB,), jnp.float32),
    mesh=_mesh,
    scratch_types=[
        pltpu.VMEM((_RPW,), jnp.int32),
        pltpu.VMEM((_BLK,), jnp.float32),
        pltpu.VMEM((_BLK,), jnp.float32),
        pltpu.VMEM((2 * _G,), jnp.int32),
        pltpu.SemaphoreType.DMA,
        pltpu.SemaphoreType.DMA,
    ],
    compiler_params=pltpu.CompilerParams(needs_layout_passes=False),
)
def _one_hot_sc(idx_hbm, out_hbm, idx_v, buf_a, buf_b, save_v, sem_a, sem_b):
    wid = lax.axis_index("s") * _NC + lax.axis_index("c")
    base = wid * _RPW

    pltpu.sync_copy(idx_hbm.at[pl.ds(base, _RPW)], idx_v)

    lanes = lax.iota(jnp.int32, _L)
    zeros = jnp.zeros((_L,), jnp.float32)
    ones = jnp.ones((_L,), jnp.float32)

    def _zero(i, c):
        for u in range(8):
            off = i * (8 * _L) + u * _L
            buf_a[pl.ds(off, _L)] = zeros
            buf_b[pl.ds(off, _L)] = zeros
        return c

    lax.fori_loop(0, _BLK // (8 * _L), _zero, jnp.int32(0))

    def step(s, buf, sem, save_off, first):
        i0 = idx_v[pl.ds(s * _G, _L)]
        i1 = idx_v[pl.ds(s * _G + _L, _L)]
        p0 = lanes * _VOCAB + i0
        p1 = (lanes + _L) * _VOCAB + i1
        if not first:
            pltpu.make_async_copy(buf, out_hbm.at[pl.ds(0, _BLK)], sem).wait()
            q0 = save_v[pl.ds(save_off, _L)]
            q1 = save_v[pl.ds(save_off + _L, _L)]
            plsc.store_scatter(buf, [q0], zeros)
            plsc.store_scatter(buf, [q1], zeros)
        plsc.store_scatter(buf, [p0], ones)
        plsc.store_scatter(buf, [p1], ones)
        save_v[pl.ds(save_off, _L)] = p0
        save_v[pl.ds(save_off + _L, _L)] = p1
        pltpu.async_copy(
            buf, out_hbm.at[pl.ds(base * _VOCAB + s * _BLK, _BLK)], sem)

    step(0, buf_a, sem_a, 0, True)
    step(1, buf_b, sem_b, _G, True)

    def _body(g, c):
        step(2 * g, buf_a, sem_a, 0, False)
        step(2 * g + 1, buf_b, sem_b, _G, False)
        return c

    lax.fori_loop(1, _STEPS // 2, _body, jnp.int32(0))

    pltpu.make_async_copy(buf_a, out_hbm.at[pl.ds(0, _BLK)], sem_a).wait()
    pltpu.make_async_copy(buf_b, out_hbm.at[pl.ds(0, _BLK)], sem_b).wait()


def kernel(inputs):
    idx = inputs.astype(jnp.int32).reshape(_ROWS)
    flat = _one_hot_sc(idx)
    return flat.reshape(inputs.shape[0], inputs.shape[1], _VOCAB)
